# baseline (speedup 1.0000x reference)
import functools

import jax
import jax.numpy as jnp
from jax import lax
from jax.experimental import pallas as pl
from jax.experimental.pallas import tpu as pltpu
from jax.experimental.pallas import tpu_sc as plsc

_info = plsc.get_sparse_core_info()
_NC, _NS = _info.num_cores, _info.num_subcores
_NW = _NC * _NS

_CH = 32
_NBUF = 4
_NTILE = 3


def _make_copy_kernel(S, D, dtype):
    rows_per_w = S // _NW
    nchunk = rows_per_w // _CH
    assert nchunk % _NBUF == 0
    mesh = plsc.VectorSubcoreMesh(core_axis_name="c", subcore_axis_name="s")

    scratch = (
        [pltpu.VMEM((_CH, D), dtype) for _ in range(_NTILE)]
        + [pltpu.VMEM_SHARED((_NS, _CH, D), dtype) for _ in range(_NBUF - _NTILE)]
        + [pltpu.SemaphoreType.DMA for _ in range(2 * _NBUF)]
    )

    @functools.partial(
        pl.kernel,
        mesh=mesh,
        out_type=jax.ShapeDtypeStruct((1, S, D), dtype),
        scratch_types=scratch,
    )
    def copy_k(table_hbm, out_hbm, *rest):
        raw_bufs = rest[:_NBUF]
        in_sems = rest[_NBUF : 2 * _NBUF]
        out_sems = rest[2 * _NBUF :]
        sid = lax.axis_index("s")
        wid = sid * _NC + lax.axis_index("c")
        base = wid * rows_per_w

        def get_buf(b):
            if b < _NTILE:
                return raw_bufs[b]
            return raw_bufs[b].at[sid]

        def start_in(g, b):
            pltpu.async_copy(
                table_hbm.at[pl.ds(base + g * _CH, _CH)], get_buf(b), in_sems[b]
            )

        def wait_in(b):
            pltpu.make_async_copy(
                table_hbm.at[pl.ds(0, _CH)], get_buf(b), in_sems[b]
            ).wait()

        def start_out(g, b):
            pltpu.async_copy(
                get_buf(b), out_hbm.at[0, pl.ds(base + g * _CH, _CH)], out_sems[b]
            )

        def wait_out(b):
            pltpu.make_async_copy(
                get_buf(b), out_hbm.at[0, pl.ds(0, _CH)], out_sems[b]
            ).wait()

        for b in range(_NBUF):
            start_in(b, b)

        @pl.loop(0, nchunk, step=_NBUF)
        def _(g0):
            for b in range(_NBUF):
                wait_in(b)
                start_out(g0 + b, b)
            for b in range(_NBUF):
                wait_out(b)

                @pl.when(g0 + b + _NBUF < nchunk)
                def _():
                    start_in(g0 + b + _NBUF, b)

    return copy_k


def kernel(x, table):
    S, D = table.shape
    return _make_copy_kernel(S, D, table.dtype)(table)


# speedup vs baseline: 1.0147x; 1.0147x over previous
import functools

import jax
import jax.numpy as jnp
from jax import lax
from jax.experimental import pallas as pl
from jax.experimental.pallas import tpu as pltpu
from jax.experimental.pallas import tpu_sc as plsc

_info = plsc.get_sparse_core_info()
_NC, _NS = _info.num_cores, _info.num_subcores
_NW = _NC * _NS

_CH = 32
_NBUF = 4
_NTILE = 1


def _make_copy_kernel(S, D, dtype):
    rows_per_w = S // _NW
    nchunk = rows_per_w // _CH
    assert nchunk % _NBUF == 0
    mesh = plsc.VectorSubcoreMesh(core_axis_name="c", subcore_axis_name="s")

    scratch = (
        [pltpu.VMEM((_CH, D), dtype) for _ in range(_NTILE)]
        + [pltpu.VMEM_SHARED((_NS, _CH, D), dtype) for _ in range(_NBUF - _NTILE)]
        + [pltpu.SemaphoreType.DMA for _ in range(2 * _NBUF)]
    )

    @functools.partial(
        pl.kernel,
        mesh=mesh,
        out_type=jax.ShapeDtypeStruct((1, S, D), dtype),
        scratch_types=scratch,
    )
    def copy_k(table_hbm, out_hbm, *rest):
        raw_bufs = rest[:_NBUF]
        in_sems = rest[_NBUF : 2 * _NBUF]
        out_sems = rest[2 * _NBUF :]
        sid = lax.axis_index("s")
        wid = sid * _NC + lax.axis_index("c")
        base = wid * rows_per_w

        def get_buf(b):
            if b < _NTILE:
                return raw_bufs[b]
            return raw_bufs[b].at[sid]

        def start_in(g, b):
            pltpu.async_copy(
                table_hbm.at[pl.ds(base + g * _CH, _CH)], get_buf(b), in_sems[b]
            )

        def wait_in(b):
            pltpu.make_async_copy(
                table_hbm.at[pl.ds(0, _CH)], get_buf(b), in_sems[b]
            ).wait()

        def start_out(g, b):
            pltpu.async_copy(
                get_buf(b), out_hbm.at[0, pl.ds(base + g * _CH, _CH)], out_sems[b]
            )

        def wait_out(b):
            pltpu.make_async_copy(
                get_buf(b), out_hbm.at[0, pl.ds(0, _CH)], out_sems[b]
            ).wait()

        for b in range(_NBUF):
            start_in(b, b)

        @pl.loop(0, nchunk, step=_NBUF)
        def _(g0):
            for b in range(_NBUF):
                wait_in(b)
                start_out(g0 + b, b)
            for b in range(_NBUF):
                wait_out(b)

                @pl.when(g0 + b + _NBUF < nchunk)
                def _():
                    start_in(g0 + b + _NBUF, b)

    return copy_k


def kernel(x, table):
    S = x.shape[1]
    D = table.shape[1]
    return _make_copy_kernel(S, D, table.dtype)(table)
